# baseline (speedup 1.0000x reference)
import functools

import jax
import jax.numpy as jnp
from jax import lax
from jax.experimental import pallas as pl
from jax.experimental.pallas import tpu as pltpu
from jax.experimental.pallas import tpu_sc as plsc

EMBED = 64
NC = 2
NS = 16
NW = NC * NS
CHUNK = 128
LANES = 16


def _emb_body(idx_hbm, table_hbm, pos2_hbm, out_hbm,
              idx_v, pos_v, rows_v, sem_in, sem_out):
    chunks_per_w = idx_hbm.shape[1]
    t = pos2_hbm.shape[0] // 2
    w = lax.axis_index("s") * NC + lax.axis_index("c")
    pltpu.sync_copy(idx_hbm.at[w], idx_v)
    pltpu.sync_copy(pos2_hbm, pos_v)
    base = w * (chunks_per_w * CHUNK)

    def chunk_body(h, _):
        pltpu.async_copy(table_hbm.at[idx_v.at[h]], rows_v, sem_in).wait()
        poff = lax.rem(h * CHUNK, t)

        def add_row(r, _):
            for c in range(EMBED // LANES):
                sl = pl.ds(c * LANES, LANES)
                rows_v[r, sl] = rows_v[r, sl] + pos_v[poff + r, sl]
            return ()

        lax.fori_loop(0, CHUNK, add_row, ())
        pltpu.async_copy(
            rows_v, out_hbm.at[pl.ds(base + h * CHUNK, CHUNK)], sem_out
        ).wait()
        return ()

    lax.fori_loop(0, chunks_per_w, chunk_body, ())


def kernel(x, token_emb, pos_emb):
    B, Tcur = x.shape
    total = B * Tcur
    chunks_per_w = total // (NW * CHUNK)
    xi = x.astype(jnp.int32).reshape(NW, chunks_per_w, CHUNK)
    pos = pos_emb[0, :Tcur, :]
    pos2 = jnp.concatenate([pos, pos], axis=0)

    emb = functools.partial(
        pl.kernel,
        out_type=jax.ShapeDtypeStruct((total, EMBED), jnp.float32),
        mesh=plsc.VectorSubcoreMesh(core_axis_name="c", subcore_axis_name="s"),
        compiler_params=pltpu.CompilerParams(use_tc_tiling_on_sc=False),
        scratch_types=[
            pltpu.VMEM((chunks_per_w, CHUNK), jnp.int32),
            pltpu.VMEM((2 * Tcur, EMBED), jnp.float32),
            pltpu.VMEM((CHUNK, EMBED), jnp.float32),
            pltpu.SemaphoreType.DMA,
            pltpu.SemaphoreType.DMA,
        ],
    )(_emb_body)
    out = emb(xi, token_emb, pos2)
    return out.reshape(B, Tcur, EMBED)


# speedup vs baseline: 1.1669x; 1.1669x over previous
import functools

import jax
import jax.numpy as jnp
from jax import lax
from jax.experimental import pallas as pl
from jax.experimental.pallas import tpu as pltpu
from jax.experimental.pallas import tpu_sc as plsc

EMBED = 64
NC = 2
NS = 16
NW = NC * NS
CHUNK = 128
NBUF = 2


def _emb_body(idx_hbm, table_hbm, pos2_hbm, out_hbm,
              idx_v, rows_v, pos_sh, sem_g, sem_s):
    chunks_per_w = idx_hbm.shape[1]
    t = pos2_hbm.shape[0] // 2
    c = lax.axis_index("c")
    s = lax.axis_index("s")
    w = s * NC + c
    pltpu.sync_copy(idx_hbm.at[w], idx_v)

    @pl.when(s == 0)
    def _():
        pltpu.sync_copy(pos2_hbm, pos_sh)
    plsc.subcore_barrier()

    base = w * (chunks_per_w * CHUNK)

    def chunk_body(h, _):
        buf = lax.rem(h, NBUF)

        @pl.when(h >= NBUF)
        def _():
            pltpu.make_async_copy(
                rows_v.at[buf], out_hbm.at[pl.ds(base, CHUNK)], sem_s
            ).wait()

        poff = lax.rem(h * CHUNK, t)
        pltpu.sync_copy(pos_sh.at[pl.ds(poff, CHUNK)], rows_v.at[buf])
        pltpu.async_copy(
            table_hbm.at[idx_v.at[h]], rows_v.at[buf], sem_g, add=True
        ).wait()
        pltpu.async_copy(
            rows_v.at[buf], out_hbm.at[pl.ds(base + h * CHUNK, CHUNK)], sem_s
        )
        return ()

    lax.fori_loop(0, chunks_per_w, chunk_body, ())

    for _ in range(NBUF):
        pltpu.make_async_copy(
            rows_v.at[0], out_hbm.at[pl.ds(base, CHUNK)], sem_s
        ).wait()


def kernel(x, token_emb, pos_emb):
    B, Tcur = x.shape
    total = B * Tcur
    chunks_per_w = total // (NW * CHUNK)
    xi = x.astype(jnp.int32).reshape(NW, chunks_per_w, CHUNK)
    pos = pos_emb[0, :Tcur, :]
    pos2 = jnp.concatenate([pos, pos], axis=0)

    emb = functools.partial(
        pl.kernel,
        out_type=jax.ShapeDtypeStruct((total, EMBED), jnp.float32),
        mesh=plsc.VectorSubcoreMesh(core_axis_name="c", subcore_axis_name="s"),
        compiler_params=pltpu.CompilerParams(use_tc_tiling_on_sc=False),
        scratch_types=[
            pltpu.VMEM((chunks_per_w, CHUNK), jnp.int32),
            pltpu.VMEM((NBUF, CHUNK, EMBED), jnp.float32),
            pltpu.VMEM_SHARED((2 * Tcur, EMBED), jnp.float32),
            pltpu.SemaphoreType.DMA,
            pltpu.SemaphoreType.DMA,
        ],
    )(_emb_body)
    out = emb(xi, token_emb, pos2)
    return out.reshape(B, Tcur, EMBED)
